# baseline (speedup 1.0000x reference)
import jax
import jax.numpy as jnp
from jax.experimental import pallas as pl

_IMG = 512.0
_CONF_T = 0.5
_IOU_T = 0.35
_NUM_PRED = 10
_NCLS = 81
_ROWS = 160
_LANES = 125

_NEG_INF = float("-inf")


def _nms_body(y_ref, o_ref):
    shape = (_ROWS, _LANES)

    conf = y_ref[0, 0]
    cls = jnp.zeros(shape, jnp.int32)
    for c in range(1, _NCLS):
        s = y_ref[0, c]
        gt = s > conf
        conf = jnp.where(gt, s, conf)
        cls = jnp.where(gt, c, cls)

    ocx = y_ref[0, 81]
    ocy = y_ref[0, 82]
    ow = y_ref[0, 83]
    oh = y_ref[0, 84]
    acx = y_ref[0, 85]
    acy = y_ref[0, 86]
    aw = y_ref[0, 87]
    ah = y_ref[0, 88]
    v0 = y_ref[0, 89]
    v1 = y_ref[0, 90]
    v2 = y_ref[0, 91]
    v3 = y_ref[0, 92]

    cx = ocx * v0 * aw + acx
    cy = ocy * v1 * ah + acy
    w = jnp.exp(ow * v2) * aw
    h = jnp.exp(oh * v3) * ah
    xmin = (cx - 0.5 * w) * _IMG
    ymin = (cy - 0.5 * h) * _IMG
    xmax = (cx + 0.5 * w) * _IMG
    ymax = (cy + 0.5 * h) * _IMG
    area = jnp.maximum(xmax - xmin, 0.0) * jnp.maximum(ymax - ymin, 0.0)

    valid = (cls != 0) & (conf >= _CONF_T)
    scores = jnp.where(valid, conf, _NEG_INF)
    clsf = cls.astype(jnp.float32)

    flat = (jax.lax.broadcasted_iota(jnp.int32, shape, 0) * _LANES
            + jax.lax.broadcasted_iota(jnp.int32, shape, 1))

    sub_i = jax.lax.broadcasted_iota(jnp.int32, (16, 128), 0)
    lane_i = jax.lax.broadcasted_iota(jnp.int32, (16, 128), 1)
    out_acc = jnp.zeros((16, 128), jnp.float32)

    for t in range(_NUM_PRED):
        m = jnp.max(scores)
        ok = m > _NEG_INF
        okf = jnp.where(ok, 1.0, 0.0).astype(jnp.float32)
        i = jnp.min(jnp.where(scores == m, flat, jnp.int32(2 ** 30)))
        sel = flat == i

        def ext(x):
            return jnp.sum(jnp.where(sel, x, 0.0))

        bcls = ext(clsf)
        bconf = ext(conf)
        bx1 = ext(xmin)
        by1 = ext(ymin)
        bx2 = ext(xmax)
        by2 = ext(ymax)

        row = (jnp.where(lane_i == 0, bcls, 0.0)
               + jnp.where(lane_i == 1, bconf, 0.0)
               + jnp.where(lane_i == 2, bx1, 0.0)
               + jnp.where(lane_i == 3, by1, 0.0)
               + jnp.where(lane_i == 4, bx2, 0.0)
               + jnp.where(lane_i == 5, by2, 0.0))
        out_acc = out_acc + okf * jnp.where(sub_i == t, row, 0.0)

        ix1 = jnp.maximum(xmin, bx1)
        iy1 = jnp.maximum(ymin, by1)
        ix2 = jnp.minimum(xmax, bx2)
        iy2 = jnp.minimum(ymax, by2)
        inter = jnp.maximum(ix2 - ix1, 0.0) * jnp.maximum(iy2 - iy1, 0.0)
        barea = (jnp.maximum(bx2 - bx1, 0.0) * jnp.maximum(by2 - by1, 0.0))
        iou = inter / jnp.maximum(area + barea - inter, 1e-8)
        supp = ((iou > _IOU_T) | sel) & ok
        scores = jnp.where(supp, _NEG_INF, scores)

    o_ref[0] = out_acc


def kernel(y_pred):
    b, n, d = y_pred.shape
    yt = jnp.transpose(y_pred, (0, 2, 1)).reshape(b, d, _ROWS, _LANES)
    out = pl.pallas_call(
        _nms_body,
        grid=(b,),
        in_specs=[pl.BlockSpec((1, d, _ROWS, _LANES),
                               lambda i: (i, 0, 0, 0))],
        out_specs=pl.BlockSpec((1, 16, 128), lambda i: (i, 0, 0)),
        out_shape=jax.ShapeDtypeStruct((b, 16, 128), jnp.float32),
    )(yt)
    return out[:, :_NUM_PRED, :6]
